# baseline (speedup 1.0000x reference)
import functools

import jax
import jax.numpy as jnp
from jax import lax
from jax.experimental import pallas as pl
from jax.experimental.pallas import tpu as pltpu
from jax.experimental.pallas import tpu_sc as plsc

N = 10000
NPAD = 10240
D = 128
DH = 64
NC = 2
NS = 16
RPT = NPAD // NS
CHUNK = 128
E_RAW = 320000
EPAD = 327680
NCHT = EPAD // CHUNK
TCH = NCHT // NS
ACH = NCHT // (NC * NS)
B = 1024
BPT = B // NS
MB = 1280
NCLS = 10

_MESH = plsc.VectorSubcoreMesh(
    core_axis_name="c", subcore_axis_name="s", num_cores=NC, num_subcores=NS)
_SC_PARAMS = pltpu.CompilerParams(use_tc_tiling_on_sc=False)


@functools.partial(
    pl.kernel,
    out_type=jax.ShapeDtypeStruct((NC, NPAD, 8), jnp.float32),
    mesh=_MESH,
    compiler_params=_SC_PARAMS,
    scratch_types=[
        pltpu.VMEM((ACH, CHUNK), jnp.int32),
        pltpu.VMEM((CHUNK, 8), jnp.float32),
        pltpu.VMEM_SHARED((NPAD, 8), jnp.float32),
        pltpu.SemaphoreType.DMA,
    ],
)
def _deg_sc(dstq_hbm, zeros_hbm, ones_hbm, cnt_hbm, dstv, onev, acc, dsem):
    c = lax.axis_index("c")
    s = lax.axis_index("s")
    r0 = s * RPT
    pltpu.sync_copy(zeros_hbm.at[pl.ds(r0, RPT)], acc.at[pl.ds(r0, RPT)])
    pltpu.sync_copy(ones_hbm, onev)
    pltpu.sync_copy(dstq_hbm.at[c, pl.ds(s * ACH, ACH)], dstv)
    plsc.subcore_barrier()

    def body(j, carry):
        pltpu.async_copy(onev, acc.at[dstv.at[j]], dsem, add=True)
        return carry

    lax.fori_loop(0, ACH, body, 0)

    def drain(j, carry):
        pltpu.make_async_copy(onev, acc.at[dstv.at[j]], dsem).wait()
        return carry

    lax.fori_loop(0, ACH, drain, 0)
    plsc.subcore_barrier()
    pltpu.sync_copy(acc.at[pl.ds(r0, RPT)], cnt_hbm.at[c, pl.ds(r0, RPT)])


NTS = 3
NHB = 2
NB = NTS + NHB
PH = 8
GPP = 4
TSP = NTS * GPP
HBP = NHB * GPP
TST = TSP * PH
HBT = HBP * PH


def _agg_edges(c, s, tf_hbm, sq_ts, sq_hb, dq_ts, dq_hb,
               tsv, hsv, tdv, hdv, bufs, ts, acc, gsem, ssem):
    r0 = s * RPT
    pltpu.sync_copy(tf_hbm.at[pl.ds(c * NPAD + r0, RPT)], ts.at[pl.ds(r0, RPT)])
    pltpu.sync_copy(tf_hbm.at[pl.ds(c * NPAD + r0, RPT)], acc.at[pl.ds(r0, RPT)])
    plsc.subcore_barrier()

    def g_src(b, o):
        if b < NTS:
            return ts.at[tsv.at[NTS * o + b]]
        return tf_hbm.at[hsv.at[NHB * o + (b - NTS)]]

    def s_dst(b, o):
        if b < NTS:
            return acc.at[tdv.at[NTS * o + b]]
        return acc.at[hdv.at[NHB * o + (b - NTS)]]

    def start_g(b, o):
        pltpu.async_copy(g_src(b, o), bufs.at[b], gsem.at[b])

    def wait_g(b, o):
        pltpu.make_async_copy(g_src(b, o), bufs.at[b], gsem.at[b]).wait()

    def start_s(b, o):
        pltpu.async_copy(bufs.at[b], s_dst(b, o), ssem.at[b], add=True)

    def wait_s(b, o):
        pltpu.make_async_copy(bufs.at[b], s_dst(b, o), ssem.at[b]).wait()

    for ph in range(PH):
        pltpu.sync_copy(sq_ts.at[pl.ds(s * TST + ph * TSP, TSP)], tsv)
        pltpu.sync_copy(sq_hb.at[c, pl.ds(s * HBT + ph * HBP, HBP)], hsv)
        pltpu.sync_copy(dq_ts.at[pl.ds(s * TST + ph * TSP, TSP)], tdv)
        pltpu.sync_copy(dq_hb.at[pl.ds(s * HBT + ph * HBP, HBP)], hdv)
        for b in range(NB):
            start_g(b, 0)

        def group(o, carry):
            for b in range(NB):
                wait_g(b, o)
                start_s(b, o)
            for b in range(NB):
                wait_s(b, o)
                start_g(b, o + 1)
            return carry

        lax.fori_loop(0, GPP - 1, group, 0)
        o = GPP - 1
        for b in range(NB):
            wait_g(b, o)
            start_s(b, o)
        for b in range(NB):
            wait_s(b, o)
    plsc.subcore_barrier()


@functools.partial(
    pl.kernel,
    out_type=jax.ShapeDtypeStruct((NC, NPAD, DH), jnp.float32),
    mesh=_MESH,
    compiler_params=_SC_PARAMS,
    scratch_types=[
        pltpu.VMEM((TSP, CHUNK), jnp.int32),
        pltpu.VMEM((HBP, CHUNK), jnp.int32),
        pltpu.VMEM((TSP, CHUNK), jnp.int32),
        pltpu.VMEM((HBP, CHUNK), jnp.int32),
        pltpu.VMEM((NB, CHUNK, DH), jnp.float32),
        pltpu.VMEM_SHARED((NPAD, DH), jnp.float32),
        pltpu.VMEM_SHARED((NPAD, DH), jnp.float32),
        pltpu.SemaphoreType.DMA((NB,)),
        pltpu.SemaphoreType.DMA((NB,)),
    ],
)
def _agg_full_sc(tf_hbm, sq_ts, sq_hb, dq_ts, dq_hb, agg_hbm,
                 tsv, hsv, tdv, hdv, bufs, ts, acc, gsem, ssem):
    c = lax.axis_index("c")
    s = lax.axis_index("s")
    _agg_edges(c, s, tf_hbm, sq_ts, sq_hb, dq_ts, dq_hb,
               tsv, hsv, tdv, hdv, bufs, ts, acc, gsem, ssem)
    r0 = s * RPT
    pltpu.sync_copy(acc.at[pl.ds(r0, RPT)], agg_hbm.at[c, pl.ds(r0, RPT)])


@functools.partial(
    pl.kernel,
    out_type=(
        jax.ShapeDtypeStruct((NC, B, DH), jnp.float32),
        jax.ShapeDtypeStruct((NC, B, 8), jnp.float32),
    ),
    mesh=_MESH,
    compiler_params=_SC_PARAMS,
    scratch_types=[
        pltpu.VMEM((TSP, CHUNK), jnp.int32),
        pltpu.VMEM((HBP, CHUNK), jnp.int32),
        pltpu.VMEM((TSP, CHUNK), jnp.int32),
        pltpu.VMEM((HBP, CHUNK), jnp.int32),
        pltpu.VMEM((NB, CHUNK, DH), jnp.float32),
        pltpu.VMEM_SHARED((NPAD, DH), jnp.float32),
        pltpu.VMEM_SHARED((NPAD, DH), jnp.float32),
        pltpu.SemaphoreType.DMA((NB,)),
        pltpu.SemaphoreType.DMA((NB,)),
        pltpu.SemaphoreType.DMA,
        pltpu.VMEM((BPT,), jnp.int32),
        pltpu.VMEM((BPT,), jnp.int32),
        pltpu.VMEM((BPT, 8), jnp.float32),
    ],
)
def _agg_batch_sc(tf_hbm, sq_ts, sq_hb, dq_ts, dq_hb, cntf_hbm, bidx2_hbm,
                  asel_hbm, csel_hbm,
                  tsv, hsv, tdv, hdv, bufs, ts, acc, gsem, ssem, sem,
                  bloc, boff, cbuf):
    c = lax.axis_index("c")
    s = lax.axis_index("s")
    _agg_edges(c, s, tf_hbm, sq_ts, sq_hb, dq_ts, dq_hb,
               tsv, hsv, tdv, hdv, bufs, ts, acc, gsem, ssem)
    b0 = s * BPT
    pltpu.sync_copy(bidx2_hbm.at[0, pl.ds(b0, BPT)], bloc)
    pltpu.sync_copy(bidx2_hbm.at[c, pl.ds(b0, BPT)], boff)
    gbuf = bufs.at[0, pl.ds(0, BPT)]
    pltpu.async_copy(acc.at[bloc], gbuf, sem).wait()
    pltpu.sync_copy(gbuf, asel_hbm.at[c, pl.ds(b0, BPT)])
    pltpu.async_copy(cntf_hbm.at[boff], cbuf, sem).wait()
    pltpu.sync_copy(cbuf, csel_hbm.at[c, pl.ds(b0, BPT)])


def _dis_from_cnt(cnt_ref):
    deg = cnt_ref[0, :, 0:1] + cnt_ref[1, :, 0:1] + 1.0
    return lax.rsqrt(deg)


def _dense1_body(x_ref, w_ref, cnt_ref, t_ref):
    u = jnp.dot(x_ref[...], w_ref[...], preferred_element_type=jnp.float32)
    us = u * _dis_from_cnt(cnt_ref)
    t_ref[0] = us[:, :DH]
    t_ref[1] = us[:, DH:]


def _dense2_body(agg_ref, cnt_ref, w_ref, b_ref, t_ref):
    dis = _dis_from_cnt(cnt_ref)
    h0 = jnp.maximum(agg_ref[0] * dis + b_ref[0:1, :DH], 0.0)
    h1 = jnp.maximum(agg_ref[1] * dis + b_ref[0:1, DH:], 0.0)
    u = (jnp.dot(h0, w_ref[:DH, :], preferred_element_type=jnp.float32)
         + jnp.dot(h1, w_ref[DH:, :], preferred_element_type=jnp.float32))
    us = u * dis
    t_ref[0] = us[:, :DH]
    t_ref[1] = us[:, DH:]


def _head_body(asel_ref, csel_ref, b2_ref, wl_ref, bl_ref,
               logp_ref, out_ref, fsel_ref):
    deg = csel_ref[0, :, 0:1] + csel_ref[1, :, 0:1] + 1.0
    dis = lax.rsqrt(deg)
    f0 = jnp.maximum(asel_ref[0] * dis + b2_ref[0:1, :DH], 0.0)
    f1 = jnp.maximum(asel_ref[1] * dis + b2_ref[0:1, DH:], 0.0)
    fsel_ref[:, :DH] = f0
    fsel_ref[:, DH:] = f1
    o = (jnp.dot(f0, wl_ref[:DH, :], preferred_element_type=jnp.float32)
         + jnp.dot(f1, wl_ref[DH:, :], preferred_element_type=jnp.float32))
    o = jnp.maximum(o + bl_ref[0:1, :], 0.0)
    out_ref[...] = o
    m = jnp.max(o, axis=1, keepdims=True)
    lse = m + jnp.log(jnp.sum(jnp.exp(o - m), axis=1, keepdims=True))
    logp_ref[...] = o - lse


_GRID = NPAD // MB

_dense1 = pl.pallas_call(
    _dense1_body,
    grid=(_GRID,),
    in_specs=[
        pl.BlockSpec((MB, D), lambda i: (i, 0)),
        pl.BlockSpec((D, D), lambda i: (0, 0)),
        pl.BlockSpec((NC, MB, 8), lambda i: (0, i, 0)),
    ],
    out_specs=pl.BlockSpec((NC, MB, DH), lambda i: (0, i, 0)),
    out_shape=jax.ShapeDtypeStruct((NC, NPAD, DH), jnp.float32),
)

_dense2 = pl.pallas_call(
    _dense2_body,
    grid=(_GRID,),
    in_specs=[
        pl.BlockSpec((NC, MB, DH), lambda i: (0, i, 0)),
        pl.BlockSpec((NC, MB, 8), lambda i: (0, i, 0)),
        pl.BlockSpec((D, D), lambda i: (0, 0)),
        pl.BlockSpec((1, D), lambda i: (0, 0)),
    ],
    out_specs=pl.BlockSpec((NC, MB, DH), lambda i: (0, i, 0)),
    out_shape=jax.ShapeDtypeStruct((NC, NPAD, DH), jnp.float32),
)

_head = pl.pallas_call(
    _head_body,
    grid=(1,),
    in_specs=[
        pl.BlockSpec((NC, B, DH), lambda i: (0, 0, 0)),
        pl.BlockSpec((NC, B, 8), lambda i: (0, 0, 0)),
        pl.BlockSpec((1, D), lambda i: (0, 0)),
        pl.BlockSpec((D, NCLS), lambda i: (0, 0)),
        pl.BlockSpec((1, NCLS), lambda i: (0, 0)),
    ],
    out_specs=(
        pl.BlockSpec((B, NCLS), lambda i: (0, 0)),
        pl.BlockSpec((B, NCLS), lambda i: (0, 0)),
        pl.BlockSpec((B, D), lambda i: (0, 0)),
    ),
    out_shape=(
        jax.ShapeDtypeStruct((B, NCLS), jnp.float32),
        jax.ShapeDtypeStruct((B, NCLS), jnp.float32),
        jax.ShapeDtypeStruct((B, D), jnp.float32),
    ),
)


def kernel(x, edge_index, batch_index, W1, b1, W2, b2, Wlin, blin):
    src = edge_index[0].astype(jnp.int32)
    dst = edge_index[1].astype(jnp.int32)
    e = src.shape[0]
    pad = EPAD - e
    srcp = jnp.concatenate([src, jnp.zeros((pad,), jnp.int32)])
    dstp = jnp.concatenate([dst, jnp.full((pad,), N, jnp.int32)])
    dstq_deg = dstp.reshape(NC, NCHT // NC, CHUNK)
    srcq3 = srcp.reshape(NS, TCH, CHUNK)
    dstq3 = dstp.reshape(NS, TCH, CHUNK)
    sq_ts = srcq3[:, :TST, :].reshape(NS * TST, CHUNK)
    hb = srcq3[:, TST:, :]
    sq_hb = jnp.stack([hb, hb + NPAD]).reshape(NC, NS * HBT, CHUNK)
    dq_ts = dstq3[:, :TST, :].reshape(NS * TST, CHUNK)
    dq_hb = dstq3[:, TST:, :].reshape(NS * HBT, CHUNK)
    xp = jnp.concatenate([x, jnp.zeros((NPAD - N, D), jnp.float32)])
    zeros8 = jnp.zeros((NPAD, 8), jnp.float32)
    ones8 = jnp.ones((CHUNK, 8), jnp.float32)

    cnt = _deg_sc(dstq_deg, zeros8, ones8)
    t1 = _dense1(xp, W1, cnt)
    agg1 = _agg_full_sc(t1.reshape(NC * NPAD, DH),
                        sq_ts, sq_hb, dq_ts, dq_hb)
    t2 = _dense2(agg1, cnt, W2, b1.reshape(1, D))
    bidx = batch_index.astype(jnp.int32)
    bidx2 = jnp.stack([bidx, bidx + NPAD])
    asel, csel = _agg_batch_sc(t2.reshape(NC * NPAD, DH),
                               sq_ts, sq_hb, dq_ts, dq_hb,
                               cnt.reshape(NC * NPAD, 8), bidx2)
    logp, out, fsel = _head(asel, csel, b2.reshape(1, D), Wlin,
                            blin.reshape(1, NCLS))
    return (logp, out, fsel)


# speedup vs baseline: 1.5313x; 1.5313x over previous
import functools

import jax
import jax.numpy as jnp
from jax import lax
from jax.experimental import pallas as pl
from jax.experimental.pallas import tpu as pltpu
from jax.experimental.pallas import tpu_sc as plsc

N = 10000
NPAD = 10240
D = 128
DH = 64
NC = 2
NS = 16
RPT = NPAD // NS
CHUNK = 128
E_RAW = 320000
EPAD = 327680
NCHT = EPAD // CHUNK
TCH = NCHT // NS
ACH = NCHT // (NC * NS)
B = 1024
BPT = B // NS
MB = 1280
NCLS = 10

_MESH = plsc.VectorSubcoreMesh(
    core_axis_name="c", subcore_axis_name="s", num_cores=NC, num_subcores=NS)
_SC_PARAMS = pltpu.CompilerParams(use_tc_tiling_on_sc=False)


@functools.partial(
    pl.kernel,
    out_type=jax.ShapeDtypeStruct((NC, NPAD, 8), jnp.float32),
    mesh=_MESH,
    compiler_params=_SC_PARAMS,
    scratch_types=[
        pltpu.VMEM((ACH, CHUNK), jnp.int32),
        pltpu.VMEM((CHUNK, 8), jnp.float32),
        pltpu.VMEM_SHARED((NPAD, 8), jnp.float32),
        pltpu.SemaphoreType.DMA,
    ],
)
def _deg_sc(dstq_hbm, zeros_hbm, ones_hbm, cnt_hbm, dstv, onev, acc, dsem):
    c = lax.axis_index("c")
    s = lax.axis_index("s")
    r0 = s * RPT
    pltpu.sync_copy(zeros_hbm.at[pl.ds(r0, RPT)], acc.at[pl.ds(r0, RPT)])
    pltpu.sync_copy(ones_hbm, onev)
    pltpu.sync_copy(dstq_hbm.at[c, pl.ds(s * ACH, ACH)], dstv)
    plsc.subcore_barrier()

    def body(j, carry):
        pltpu.async_copy(onev, acc.at[dstv.at[j]], dsem, add=True)
        return carry

    lax.fori_loop(0, ACH, body, 0)

    def drain(j, carry):
        pltpu.make_async_copy(onev, acc.at[dstv.at[j]], dsem).wait()
        return carry

    lax.fori_loop(0, ACH, drain, 0)
    plsc.subcore_barrier()
    pltpu.sync_copy(acc.at[pl.ds(r0, RPT)], cnt_hbm.at[c, pl.ds(r0, RPT)])


NB = 2
TCH2 = TCH // 2


def _agg_edges(c, s, tf_hbm, srcq_hbm, dstq_hbm, srcv, dstv, bufs, ts, acc,
               gsem, ssem):
    r0 = s * RPT
    pltpu.sync_copy(tf_hbm.at[pl.ds(c * NPAD + r0, RPT)], ts.at[pl.ds(r0, RPT)])
    pltpu.sync_copy(tf_hbm.at[pl.ds(c * NPAD + r0, RPT)], acc.at[pl.ds(r0, RPT)])
    plsc.subcore_barrier()

    def start_g(b, j):
        pltpu.async_copy(ts.at[srcv.at[j]], bufs.at[b], gsem.at[b])

    def wait_g(b, j):
        pltpu.make_async_copy(ts.at[srcv.at[j]], bufs.at[b],
                              gsem.at[b]).wait()

    def start_s(b, j):
        pltpu.async_copy(bufs.at[b], acc.at[dstv.at[j]], ssem.at[b], add=True)

    def wait_s(b, j):
        pltpu.make_async_copy(bufs.at[b], acc.at[dstv.at[j]],
                              ssem.at[b]).wait()

    for ph in range(2):
        q0 = s * TCH + ph * TCH2
        pltpu.sync_copy(srcq_hbm.at[pl.ds(q0, TCH2)], srcv)
        pltpu.sync_copy(dstq_hbm.at[pl.ds(q0, TCH2)], dstv)
        for b in range(NB):
            start_g(b, b)

        def outer(o, carry):
            j0 = o * NB
            for b in range(NB):
                wait_g(b, j0 + b)
                start_s(b, j0 + b)
            for b in range(NB):
                wait_s(b, j0 + b)
                start_g(b, j0 + NB + b)
            return carry

        lax.fori_loop(0, TCH2 // NB - 1, outer, 0)
        j0 = TCH2 - NB
        for b in range(NB):
            wait_g(b, j0 + b)
            start_s(b, j0 + b)
        for b in range(NB):
            wait_s(b, j0 + b)
    plsc.subcore_barrier()


@functools.partial(
    pl.kernel,
    out_type=jax.ShapeDtypeStruct((NC, NPAD, DH), jnp.float32),
    mesh=_MESH,
    compiler_params=_SC_PARAMS,
    scratch_types=[
        pltpu.VMEM((TCH2, CHUNK), jnp.int32),
        pltpu.VMEM((TCH2, CHUNK), jnp.int32),
        pltpu.VMEM((NB, CHUNK, DH), jnp.float32),
        pltpu.VMEM_SHARED((NPAD, DH), jnp.float32),
        pltpu.VMEM_SHARED((NPAD, DH), jnp.float32),
        pltpu.SemaphoreType.DMA((NB,)),
        pltpu.SemaphoreType.DMA((NB,)),
    ],
)
def _agg_full_sc(tf_hbm, srcq_hbm, dstq_hbm, agg_hbm, srcv, dstv, bufs, ts,
                 acc, gsem, ssem):
    c = lax.axis_index("c")
    s = lax.axis_index("s")
    _agg_edges(c, s, tf_hbm, srcq_hbm, dstq_hbm, srcv, dstv, bufs, ts, acc,
               gsem, ssem)
    r0 = s * RPT
    pltpu.sync_copy(acc.at[pl.ds(r0, RPT)], agg_hbm.at[c, pl.ds(r0, RPT)])


@functools.partial(
    pl.kernel,
    out_type=(
        jax.ShapeDtypeStruct((NC, B, DH), jnp.float32),
        jax.ShapeDtypeStruct((NC, B, 8), jnp.float32),
    ),
    mesh=_MESH,
    compiler_params=_SC_PARAMS,
    scratch_types=[
        pltpu.VMEM((TCH2, CHUNK), jnp.int32),
        pltpu.VMEM((TCH2, CHUNK), jnp.int32),
        pltpu.VMEM((NB, CHUNK, DH), jnp.float32),
        pltpu.VMEM_SHARED((NPAD, DH), jnp.float32),
        pltpu.VMEM_SHARED((NPAD, DH), jnp.float32),
        pltpu.SemaphoreType.DMA((NB,)),
        pltpu.SemaphoreType.DMA((NB,)),
        pltpu.SemaphoreType.DMA,
        pltpu.VMEM((BPT,), jnp.int32),
        pltpu.VMEM((BPT,), jnp.int32),
        pltpu.VMEM((BPT, DH), jnp.float32),
        pltpu.VMEM((BPT, 8), jnp.float32),
    ],
)
def _agg_batch_sc(tf_hbm, srcq_hbm, dstq_hbm, cntf_hbm, bidx2_hbm,
                  asel_hbm, csel_hbm,
                  srcv, dstv, bufs, ts, acc, gsem, ssem, sem, bloc, boff,
                  gbuf, cbuf):
    c = lax.axis_index("c")
    s = lax.axis_index("s")
    _agg_edges(c, s, tf_hbm, srcq_hbm, dstq_hbm, srcv, dstv, bufs, ts, acc,
               gsem, ssem)
    b0 = s * BPT
    pltpu.sync_copy(bidx2_hbm.at[0, pl.ds(b0, BPT)], bloc)
    pltpu.sync_copy(bidx2_hbm.at[c, pl.ds(b0, BPT)], boff)
    pltpu.async_copy(acc.at[bloc], gbuf, sem).wait()
    pltpu.sync_copy(gbuf, asel_hbm.at[c, pl.ds(b0, BPT)])
    pltpu.async_copy(cntf_hbm.at[boff], cbuf, sem).wait()
    pltpu.sync_copy(cbuf, csel_hbm.at[c, pl.ds(b0, BPT)])


def _dis_from_cnt(cnt_ref):
    deg = cnt_ref[0, :, 0:1] + cnt_ref[1, :, 0:1] + 1.0
    return lax.rsqrt(deg)


def _dense1_body(x_ref, w_ref, cnt_ref, t_ref):
    u = jnp.dot(x_ref[...], w_ref[...], preferred_element_type=jnp.float32)
    us = u * _dis_from_cnt(cnt_ref)
    t_ref[0] = us[:, :DH]
    t_ref[1] = us[:, DH:]


def _dense2_body(agg_ref, cnt_ref, w_ref, b_ref, t_ref):
    dis = _dis_from_cnt(cnt_ref)
    h0 = jnp.maximum(agg_ref[0] * dis + b_ref[0:1, :DH], 0.0)
    h1 = jnp.maximum(agg_ref[1] * dis + b_ref[0:1, DH:], 0.0)
    u = (jnp.dot(h0, w_ref[:DH, :], preferred_element_type=jnp.float32)
         + jnp.dot(h1, w_ref[DH:, :], preferred_element_type=jnp.float32))
    us = u * dis
    t_ref[0] = us[:, :DH]
    t_ref[1] = us[:, DH:]


def _head_body(asel_ref, csel_ref, b2_ref, wl_ref, bl_ref,
               logp_ref, out_ref, fsel_ref):
    deg = csel_ref[0, :, 0:1] + csel_ref[1, :, 0:1] + 1.0
    dis = lax.rsqrt(deg)
    f0 = jnp.maximum(asel_ref[0] * dis + b2_ref[0:1, :DH], 0.0)
    f1 = jnp.maximum(asel_ref[1] * dis + b2_ref[0:1, DH:], 0.0)
    fsel_ref[:, :DH] = f0
    fsel_ref[:, DH:] = f1
    o = (jnp.dot(f0, wl_ref[:DH, :], preferred_element_type=jnp.float32)
         + jnp.dot(f1, wl_ref[DH:, :], preferred_element_type=jnp.float32))
    o = jnp.maximum(o + bl_ref[0:1, :], 0.0)
    out_ref[...] = o
    m = jnp.max(o, axis=1, keepdims=True)
    lse = m + jnp.log(jnp.sum(jnp.exp(o - m), axis=1, keepdims=True))
    logp_ref[...] = o - lse


_GRID = NPAD // MB

_dense1 = pl.pallas_call(
    _dense1_body,
    grid=(_GRID,),
    in_specs=[
        pl.BlockSpec((MB, D), lambda i: (i, 0)),
        pl.BlockSpec((D, D), lambda i: (0, 0)),
        pl.BlockSpec((NC, MB, 8), lambda i: (0, i, 0)),
    ],
    out_specs=pl.BlockSpec((NC, MB, DH), lambda i: (0, i, 0)),
    out_shape=jax.ShapeDtypeStruct((NC, NPAD, DH), jnp.float32),
)

_dense2 = pl.pallas_call(
    _dense2_body,
    grid=(_GRID,),
    in_specs=[
        pl.BlockSpec((NC, MB, DH), lambda i: (0, i, 0)),
        pl.BlockSpec((NC, MB, 8), lambda i: (0, i, 0)),
        pl.BlockSpec((D, D), lambda i: (0, 0)),
        pl.BlockSpec((1, D), lambda i: (0, 0)),
    ],
    out_specs=pl.BlockSpec((NC, MB, DH), lambda i: (0, i, 0)),
    out_shape=jax.ShapeDtypeStruct((NC, NPAD, DH), jnp.float32),
)

_head = pl.pallas_call(
    _head_body,
    grid=(1,),
    in_specs=[
        pl.BlockSpec((NC, B, DH), lambda i: (0, 0, 0)),
        pl.BlockSpec((NC, B, 8), lambda i: (0, 0, 0)),
        pl.BlockSpec((1, D), lambda i: (0, 0)),
        pl.BlockSpec((D, NCLS), lambda i: (0, 0)),
        pl.BlockSpec((1, NCLS), lambda i: (0, 0)),
    ],
    out_specs=(
        pl.BlockSpec((B, NCLS), lambda i: (0, 0)),
        pl.BlockSpec((B, NCLS), lambda i: (0, 0)),
        pl.BlockSpec((B, D), lambda i: (0, 0)),
    ),
    out_shape=(
        jax.ShapeDtypeStruct((B, NCLS), jnp.float32),
        jax.ShapeDtypeStruct((B, NCLS), jnp.float32),
        jax.ShapeDtypeStruct((B, D), jnp.float32),
    ),
)


def kernel(x, edge_index, batch_index, W1, b1, W2, b2, Wlin, blin):
    src = edge_index[0].astype(jnp.int32)
    dst = edge_index[1].astype(jnp.int32)
    e = src.shape[0]
    pad = EPAD - e
    srcp = jnp.concatenate([src, jnp.zeros((pad,), jnp.int32)])
    dstp = jnp.concatenate([dst, jnp.full((pad,), N, jnp.int32)])
    srcq = srcp.reshape(NCHT, CHUNK)
    dstq_deg = dstp.reshape(NC, NCHT // NC, CHUNK)
    dstq = dstp.reshape(NCHT, CHUNK)
    xp = jnp.concatenate([x, jnp.zeros((NPAD - N, D), jnp.float32)])
    zeros8 = jnp.zeros((NPAD, 8), jnp.float32)
    ones8 = jnp.ones((CHUNK, 8), jnp.float32)

    cnt = _deg_sc(dstq_deg, zeros8, ones8)
    t1 = _dense1(xp, W1, cnt)
    agg1 = _agg_full_sc(t1.reshape(NC * NPAD, DH), srcq, dstq)
    t2 = _dense2(agg1, cnt, W2, b1.reshape(1, D))
    bidx = batch_index.astype(jnp.int32)
    bidx2 = jnp.stack([bidx, bidx + NPAD])
    asel, csel = _agg_batch_sc(t2.reshape(NC * NPAD, DH), srcq, dstq,
                               cnt.reshape(NC * NPAD, 8), bidx2)
    logp, out, fsel = _head(asel, csel, b2.reshape(1, D), Wlin,
                            blin.reshape(1, NCLS))
    return (logp, out, fsel)


# speedup vs baseline: 1.5654x; 1.0222x over previous
import functools

import jax
import jax.numpy as jnp
from jax import lax
from jax.experimental import pallas as pl
from jax.experimental.pallas import tpu as pltpu
from jax.experimental.pallas import tpu_sc as plsc

N = 10000
NPAD = 10240
D = 128
DH = 64
NC = 2
NS = 16
RPT = NPAD // NS
CHUNK = 128
E_RAW = 320000
EPAD = 327680
NCHT = EPAD // CHUNK
TCH = NCHT // NS
ACH = NCHT // (NC * NS)
B = 1024
BPT = B // NS
MB = 1280
NCLS = 10

_MESH = plsc.VectorSubcoreMesh(
    core_axis_name="c", subcore_axis_name="s", num_cores=NC, num_subcores=NS)
_SC_PARAMS = pltpu.CompilerParams(use_tc_tiling_on_sc=False)


@functools.partial(
    pl.kernel,
    out_type=jax.ShapeDtypeStruct((NC, NPAD, 8), jnp.float32),
    mesh=_MESH,
    compiler_params=_SC_PARAMS,
    scratch_types=[
        pltpu.VMEM((ACH, CHUNK), jnp.int32),
        pltpu.VMEM((CHUNK, 8), jnp.float32),
        pltpu.VMEM_SHARED((NPAD, 8), jnp.float32),
        pltpu.SemaphoreType.DMA,
    ],
)
def _deg_sc(dstq_hbm, zeros_hbm, ones_hbm, cnt_hbm, dstv, onev, acc, dsem):
    c = lax.axis_index("c")
    s = lax.axis_index("s")
    r0 = s * RPT
    inits = [
        pltpu.async_copy(zeros_hbm.at[pl.ds(r0, RPT)], acc.at[pl.ds(r0, RPT)],
                         dsem),
        pltpu.async_copy(ones_hbm, onev, dsem),
        pltpu.async_copy(dstq_hbm.at[c, pl.ds(s * ACH, ACH)], dstv, dsem),
    ]
    for a in inits:
        a.wait()
    plsc.subcore_barrier()

    def body(j, carry):
        pltpu.async_copy(onev, acc.at[dstv.at[j]], dsem, add=True)
        return carry

    lax.fori_loop(0, ACH, body, 0)

    def drain(j, carry):
        pltpu.make_async_copy(onev, acc.at[dstv.at[j]], dsem).wait()
        return carry

    lax.fori_loop(0, ACH, drain, 0)
    plsc.subcore_barrier()
    pltpu.sync_copy(acc.at[pl.ds(r0, RPT)], cnt_hbm.at[c, pl.ds(r0, RPT)])


NB = 2
TCH2 = TCH // 2


def _agg_edges(c, s, tf_hbm, srcq_hbm, dstq_hbm, srcv, dstv, bufs, ts, acc,
               gsem, ssem):
    r0 = s * RPT
    inits = [
        pltpu.async_copy(tf_hbm.at[pl.ds(c * NPAD + r0, RPT)],
                         ts.at[pl.ds(r0, RPT)], gsem.at[0]),
        pltpu.async_copy(tf_hbm.at[pl.ds(c * NPAD + r0, RPT)],
                         acc.at[pl.ds(r0, RPT)], gsem.at[1]),
        pltpu.async_copy(srcq_hbm.at[pl.ds(s * TCH, TCH2)], srcv, ssem.at[0]),
        pltpu.async_copy(dstq_hbm.at[pl.ds(s * TCH, TCH2)], dstv, ssem.at[1]),
    ]
    for a in inits:
        a.wait()
    plsc.subcore_barrier()

    def start_g(b, j):
        pltpu.async_copy(ts.at[srcv.at[j]], bufs.at[b], gsem.at[b])

    def wait_g(b, j):
        pltpu.make_async_copy(ts.at[srcv.at[j]], bufs.at[b],
                              gsem.at[b]).wait()

    def start_s(b, j):
        pltpu.async_copy(bufs.at[b], acc.at[dstv.at[j]], ssem.at[b], add=True)

    def wait_s(b, j):
        pltpu.make_async_copy(bufs.at[b], acc.at[dstv.at[j]],
                              ssem.at[b]).wait()

    for ph in range(2):
        if ph > 0:
            q0 = s * TCH + ph * TCH2
            a1 = pltpu.async_copy(srcq_hbm.at[pl.ds(q0, TCH2)], srcv,
                                  gsem.at[0])
            a2 = pltpu.async_copy(dstq_hbm.at[pl.ds(q0, TCH2)], dstv,
                                  gsem.at[1])
            a1.wait()
            a2.wait()
        for b in range(NB):
            start_g(b, b)

        def outer(o, carry):
            j0 = o * NB
            for b in range(NB):
                wait_g(b, j0 + b)
                start_s(b, j0 + b)
            for b in range(NB):
                wait_s(b, j0 + b)
                start_g(b, j0 + NB + b)
            return carry

        lax.fori_loop(0, TCH2 // NB - 1, outer, 0)
        j0 = TCH2 - NB
        for b in range(NB):
            wait_g(b, j0 + b)
            start_s(b, j0 + b)
        for b in range(NB):
            wait_s(b, j0 + b)
    plsc.subcore_barrier()


@functools.partial(
    pl.kernel,
    out_type=jax.ShapeDtypeStruct((NC, NPAD, DH), jnp.float32),
    mesh=_MESH,
    compiler_params=_SC_PARAMS,
    scratch_types=[
        pltpu.VMEM((TCH2, CHUNK), jnp.int32),
        pltpu.VMEM((TCH2, CHUNK), jnp.int32),
        pltpu.VMEM((NB, CHUNK, DH), jnp.float32),
        pltpu.VMEM_SHARED((NPAD, DH), jnp.float32),
        pltpu.VMEM_SHARED((NPAD, DH), jnp.float32),
        pltpu.SemaphoreType.DMA((NB,)),
        pltpu.SemaphoreType.DMA((NB,)),
    ],
)
def _agg_full_sc(tf_hbm, srcq_hbm, dstq_hbm, agg_hbm, srcv, dstv, bufs, ts,
                 acc, gsem, ssem):
    c = lax.axis_index("c")
    s = lax.axis_index("s")
    _agg_edges(c, s, tf_hbm, srcq_hbm, dstq_hbm, srcv, dstv, bufs, ts, acc,
               gsem, ssem)
    r0 = s * RPT
    pltpu.sync_copy(acc.at[pl.ds(r0, RPT)], agg_hbm.at[c, pl.ds(r0, RPT)])


@functools.partial(
    pl.kernel,
    out_type=(
        jax.ShapeDtypeStruct((NC, B, DH), jnp.float32),
        jax.ShapeDtypeStruct((NC, B, 8), jnp.float32),
    ),
    mesh=_MESH,
    compiler_params=_SC_PARAMS,
    scratch_types=[
        pltpu.VMEM((TCH2, CHUNK), jnp.int32),
        pltpu.VMEM((TCH2, CHUNK), jnp.int32),
        pltpu.VMEM((NB, CHUNK, DH), jnp.float32),
        pltpu.VMEM_SHARED((NPAD, DH), jnp.float32),
        pltpu.VMEM_SHARED((NPAD, DH), jnp.float32),
        pltpu.SemaphoreType.DMA((NB,)),
        pltpu.SemaphoreType.DMA((NB,)),
        pltpu.SemaphoreType.DMA,
        pltpu.VMEM((BPT,), jnp.int32),
        pltpu.VMEM((BPT,), jnp.int32),
        pltpu.VMEM((BPT, DH), jnp.float32),
        pltpu.VMEM((BPT, 8), jnp.float32),
    ],
)
def _agg_batch_sc(tf_hbm, srcq_hbm, dstq_hbm, cntf_hbm, bidx2_hbm,
                  asel_hbm, csel_hbm,
                  srcv, dstv, bufs, ts, acc, gsem, ssem, sem, bloc, boff,
                  gbuf, cbuf):
    c = lax.axis_index("c")
    s = lax.axis_index("s")
    _agg_edges(c, s, tf_hbm, srcq_hbm, dstq_hbm, srcv, dstv, bufs, ts, acc,
               gsem, ssem)
    b0 = s * BPT
    pltpu.sync_copy(bidx2_hbm.at[0, pl.ds(b0, BPT)], bloc)
    pltpu.sync_copy(bidx2_hbm.at[c, pl.ds(b0, BPT)], boff)
    pltpu.async_copy(acc.at[bloc], gbuf, sem).wait()
    pltpu.sync_copy(gbuf, asel_hbm.at[c, pl.ds(b0, BPT)])
    pltpu.async_copy(cntf_hbm.at[boff], cbuf, sem).wait()
    pltpu.sync_copy(cbuf, csel_hbm.at[c, pl.ds(b0, BPT)])


def _dis_from_cnt(cnt_ref):
    deg = cnt_ref[0, :, 0:1] + cnt_ref[1, :, 0:1] + 1.0
    return lax.rsqrt(deg)


def _dense1_body(x_ref, w_ref, cnt_ref, t_ref):
    u = jnp.dot(x_ref[...], w_ref[...], preferred_element_type=jnp.float32)
    us = u * _dis_from_cnt(cnt_ref)
    t_ref[0] = us[:, :DH]
    t_ref[1] = us[:, DH:]


def _dense2_body(agg_ref, cnt_ref, w_ref, b_ref, t_ref):
    dis = _dis_from_cnt(cnt_ref)
    h0 = jnp.maximum(agg_ref[0] * dis + b_ref[0:1, :DH], 0.0)
    h1 = jnp.maximum(agg_ref[1] * dis + b_ref[0:1, DH:], 0.0)
    u = (jnp.dot(h0, w_ref[:DH, :], preferred_element_type=jnp.float32)
         + jnp.dot(h1, w_ref[DH:, :], preferred_element_type=jnp.float32))
    us = u * dis
    t_ref[0] = us[:, :DH]
    t_ref[1] = us[:, DH:]


def _head_body(asel_ref, csel_ref, b2_ref, wl_ref, bl_ref,
               logp_ref, out_ref, fsel_ref):
    deg = csel_ref[0, :, 0:1] + csel_ref[1, :, 0:1] + 1.0
    dis = lax.rsqrt(deg)
    f0 = jnp.maximum(asel_ref[0] * dis + b2_ref[0:1, :DH], 0.0)
    f1 = jnp.maximum(asel_ref[1] * dis + b2_ref[0:1, DH:], 0.0)
    fsel_ref[:, :DH] = f0
    fsel_ref[:, DH:] = f1
    o = (jnp.dot(f0, wl_ref[:DH, :], preferred_element_type=jnp.float32)
         + jnp.dot(f1, wl_ref[DH:, :], preferred_element_type=jnp.float32))
    o = jnp.maximum(o + bl_ref[0:1, :], 0.0)
    out_ref[...] = o
    m = jnp.max(o, axis=1, keepdims=True)
    lse = m + jnp.log(jnp.sum(jnp.exp(o - m), axis=1, keepdims=True))
    logp_ref[...] = o - lse


_GRID = NPAD // MB

_dense1 = pl.pallas_call(
    _dense1_body,
    grid=(_GRID,),
    in_specs=[
        pl.BlockSpec((MB, D), lambda i: (i, 0)),
        pl.BlockSpec((D, D), lambda i: (0, 0)),
        pl.BlockSpec((NC, MB, 8), lambda i: (0, i, 0)),
    ],
    out_specs=pl.BlockSpec((NC, MB, DH), lambda i: (0, i, 0)),
    out_shape=jax.ShapeDtypeStruct((NC, NPAD, DH), jnp.float32),
)

_dense2 = pl.pallas_call(
    _dense2_body,
    grid=(_GRID,),
    in_specs=[
        pl.BlockSpec((NC, MB, DH), lambda i: (0, i, 0)),
        pl.BlockSpec((NC, MB, 8), lambda i: (0, i, 0)),
        pl.BlockSpec((D, D), lambda i: (0, 0)),
        pl.BlockSpec((1, D), lambda i: (0, 0)),
    ],
    out_specs=pl.BlockSpec((NC, MB, DH), lambda i: (0, i, 0)),
    out_shape=jax.ShapeDtypeStruct((NC, NPAD, DH), jnp.float32),
)

_head = pl.pallas_call(
    _head_body,
    grid=(1,),
    in_specs=[
        pl.BlockSpec((NC, B, DH), lambda i: (0, 0, 0)),
        pl.BlockSpec((NC, B, 8), lambda i: (0, 0, 0)),
        pl.BlockSpec((1, D), lambda i: (0, 0)),
        pl.BlockSpec((D, NCLS), lambda i: (0, 0)),
        pl.BlockSpec((1, NCLS), lambda i: (0, 0)),
    ],
    out_specs=(
        pl.BlockSpec((B, NCLS), lambda i: (0, 0)),
        pl.BlockSpec((B, NCLS), lambda i: (0, 0)),
        pl.BlockSpec((B, D), lambda i: (0, 0)),
    ),
    out_shape=(
        jax.ShapeDtypeStruct((B, NCLS), jnp.float32),
        jax.ShapeDtypeStruct((B, NCLS), jnp.float32),
        jax.ShapeDtypeStruct((B, D), jnp.float32),
    ),
)


def kernel(x, edge_index, batch_index, W1, b1, W2, b2, Wlin, blin):
    src = edge_index[0].astype(jnp.int32)
    dst = edge_index[1].astype(jnp.int32)
    e = src.shape[0]
    pad = EPAD - e
    srcp = jnp.concatenate([src, jnp.zeros((pad,), jnp.int32)])
    dstp = jnp.concatenate([dst, jnp.full((pad,), N, jnp.int32)])
    srcq = srcp.reshape(NCHT, CHUNK)
    dstq_deg = dstp.reshape(NC, NCHT // NC, CHUNK)
    dstq = dstp.reshape(NCHT, CHUNK)
    xp = jnp.concatenate([x, jnp.zeros((NPAD - N, D), jnp.float32)])
    zeros8 = jnp.zeros((NPAD, 8), jnp.float32)
    ones8 = jnp.ones((CHUNK, 8), jnp.float32)

    cnt = _deg_sc(dstq_deg, zeros8, ones8)
    t1 = _dense1(xp, W1, cnt)
    agg1 = _agg_full_sc(t1.reshape(NC * NPAD, DH), srcq, dstq)
    t2 = _dense2(agg1, cnt, W2, b1.reshape(1, D))
    bidx = batch_index.astype(jnp.int32)
    bidx2 = jnp.stack([bidx, bidx + NPAD])
    asel, csel = _agg_batch_sc(t2.reshape(NC * NPAD, DH), srcq, dstq,
                               cnt.reshape(NC * NPAD, 8), bidx2)
    logp, out, fsel = _head(asel, csel, b2.reshape(1, D), Wlin,
                            blin.reshape(1, NCLS))
    return (logp, out, fsel)


# speedup vs baseline: 1.5657x; 1.0002x over previous
import functools

import jax
import jax.numpy as jnp
from jax import lax
from jax.experimental import pallas as pl
from jax.experimental.pallas import tpu as pltpu
from jax.experimental.pallas import tpu_sc as plsc

N = 10000
NPAD = 10240
D = 128
DH = 64
NC = 2
NS = 16
RPT = NPAD // NS
CHUNK = 128
E_RAW = 320000
EPAD = 327680
NCHT = EPAD // CHUNK
TCH = NCHT // NS
ACH = NCHT // (NC * NS)
B = 1024
BPT = B // NS
MB = 1280
NCLS = 10

_MESH = plsc.VectorSubcoreMesh(
    core_axis_name="c", subcore_axis_name="s", num_cores=NC, num_subcores=NS)
_SC_PARAMS = pltpu.CompilerParams(use_tc_tiling_on_sc=False)


@functools.partial(
    pl.kernel,
    out_type=jax.ShapeDtypeStruct((NC, NPAD, 8), jnp.float32),
    mesh=_MESH,
    compiler_params=_SC_PARAMS,
    scratch_types=[
        pltpu.VMEM((ACH, CHUNK), jnp.int32),
        pltpu.VMEM((CHUNK, 8), jnp.float32),
        pltpu.VMEM_SHARED((NPAD, 8), jnp.float32),
        pltpu.SemaphoreType.DMA,
    ],
)
def _deg_sc(dstq_hbm, zeros_hbm, ones_hbm, cnt_hbm, dstv, onev, acc, dsem):
    c = lax.axis_index("c")
    s = lax.axis_index("s")
    r0 = s * RPT
    inits = [
        pltpu.async_copy(zeros_hbm.at[pl.ds(r0, RPT)], acc.at[pl.ds(r0, RPT)],
                         dsem),
        pltpu.async_copy(ones_hbm, onev, dsem),
        pltpu.async_copy(dstq_hbm.at[c, pl.ds(s * ACH, ACH)], dstv, dsem),
    ]
    for a in inits:
        a.wait()
    plsc.subcore_barrier()

    def body(j, carry):
        pltpu.async_copy(onev, acc.at[dstv.at[j]], dsem, add=True)
        return carry

    lax.fori_loop(0, ACH, body, 0)

    def drain(j, carry):
        pltpu.make_async_copy(onev, acc.at[dstv.at[j]], dsem).wait()
        return carry

    lax.fori_loop(0, ACH, drain, 0)
    plsc.subcore_barrier()
    pltpu.sync_copy(acc.at[pl.ds(r0, RPT)], cnt_hbm.at[c, pl.ds(r0, RPT)])


NB = 2
TCH2 = TCH // 2


def _agg_edges(c, s, tf_hbm, srcq_hbm, dstq_hbm, srcv, dstv, bufs, ts, acc,
               gsem, ssem):
    r0 = s * RPT
    inits = [
        pltpu.async_copy(tf_hbm.at[pl.ds(c * NPAD + r0, RPT)],
                         ts.at[pl.ds(r0, RPT)], gsem.at[0]),
        pltpu.async_copy(tf_hbm.at[pl.ds(c * NPAD + r0, RPT)],
                         acc.at[pl.ds(r0, RPT)], gsem.at[1]),
        pltpu.async_copy(srcq_hbm.at[pl.ds(s * TCH, TCH2)], srcv, ssem.at[0]),
        pltpu.async_copy(dstq_hbm.at[pl.ds(s * TCH, TCH2)], dstv, ssem.at[1]),
    ]
    for a in inits:
        a.wait()
    plsc.subcore_barrier()

    def start_g(b, j):
        pltpu.async_copy(ts.at[srcv.at[j]], bufs.at[b], gsem.at[b])

    def wait_g(b, j):
        pltpu.make_async_copy(ts.at[srcv.at[j]], bufs.at[b],
                              gsem.at[b]).wait()

    def start_s(b, j):
        pltpu.async_copy(bufs.at[b], acc.at[dstv.at[j]], ssem.at[b], add=True)

    def wait_s(b, j):
        pltpu.make_async_copy(bufs.at[b], acc.at[dstv.at[j]],
                              ssem.at[b]).wait()

    for ph in range(2):
        if ph > 0:
            q0 = s * TCH + ph * TCH2
            a1 = pltpu.async_copy(srcq_hbm.at[pl.ds(q0, TCH2)], srcv,
                                  gsem.at[0])
            a2 = pltpu.async_copy(dstq_hbm.at[pl.ds(q0, TCH2)], dstv,
                                  gsem.at[1])
            a1.wait()
            a2.wait()
        for b in range(NB):
            start_g(b, b)

        def outer(o, carry):
            j0 = o * NB
            for b in range(NB):
                wait_g(b, j0 + b)
                start_s(b, j0 + b)
            for b in range(NB):
                wait_s(b, j0 + b)
                start_g(b, j0 + NB + b)
            return carry

        lax.fori_loop(0, TCH2 // NB - 1, outer, 0)
        j0 = TCH2 - NB
        for b in range(NB):
            wait_g(b, j0 + b)
            start_s(b, j0 + b)
        for b in range(NB):
            wait_s(b, j0 + b)
    plsc.subcore_barrier()


@functools.partial(
    pl.kernel,
    out_type=jax.ShapeDtypeStruct((NC, NPAD, DH), jnp.float32),
    mesh=_MESH,
    compiler_params=_SC_PARAMS,
    scratch_types=[
        pltpu.VMEM((TCH2, CHUNK), jnp.int32),
        pltpu.VMEM((TCH2, CHUNK), jnp.int32),
        pltpu.VMEM((NB, CHUNK, DH), jnp.float32),
        pltpu.VMEM_SHARED((NPAD, DH), jnp.float32),
        pltpu.VMEM_SHARED((NPAD, DH), jnp.float32),
        pltpu.SemaphoreType.DMA((NB,)),
        pltpu.SemaphoreType.DMA((NB,)),
    ],
)
def _agg_full_sc(tf_hbm, srcq_hbm, dstq_hbm, agg_hbm, srcv, dstv, bufs, ts,
                 acc, gsem, ssem):
    c = lax.axis_index("c")
    s = lax.axis_index("s")
    _agg_edges(c, s, tf_hbm, srcq_hbm, dstq_hbm, srcv, dstv, bufs, ts, acc,
               gsem, ssem)
    r0 = s * RPT
    pltpu.sync_copy(acc.at[pl.ds(r0, RPT)], agg_hbm.at[c, pl.ds(r0, RPT)])


@functools.partial(
    pl.kernel,
    out_type=(
        jax.ShapeDtypeStruct((NC, B, DH), jnp.float32),
        jax.ShapeDtypeStruct((NC, B, 8), jnp.float32),
    ),
    mesh=_MESH,
    compiler_params=_SC_PARAMS,
    scratch_types=[
        pltpu.VMEM((TCH2, CHUNK), jnp.int32),
        pltpu.VMEM((TCH2, CHUNK), jnp.int32),
        pltpu.VMEM((NB, CHUNK, DH), jnp.float32),
        pltpu.VMEM_SHARED((NPAD, DH), jnp.float32),
        pltpu.VMEM_SHARED((NPAD, DH), jnp.float32),
        pltpu.SemaphoreType.DMA((NB,)),
        pltpu.SemaphoreType.DMA((NB,)),
        pltpu.SemaphoreType.DMA,
        pltpu.VMEM((BPT,), jnp.int32),
        pltpu.VMEM((BPT,), jnp.int32),
        pltpu.VMEM((BPT, DH), jnp.float32),
        pltpu.VMEM((BPT, 8), jnp.float32),
    ],
)
def _agg_batch_sc(tf_hbm, srcq_hbm, dstq_hbm, cntf_hbm, bidx2_hbm,
                  asel_hbm, csel_hbm,
                  srcv, dstv, bufs, ts, acc, gsem, ssem, sem, bloc, boff,
                  gbuf, cbuf):
    c = lax.axis_index("c")
    s = lax.axis_index("s")
    _agg_edges(c, s, tf_hbm, srcq_hbm, dstq_hbm, srcv, dstv, bufs, ts, acc,
               gsem, ssem)
    b0 = s * BPT
    i1 = pltpu.async_copy(bidx2_hbm.at[0, pl.ds(b0, BPT)], bloc, gsem.at[0])
    i2 = pltpu.async_copy(bidx2_hbm.at[c, pl.ds(b0, BPT)], boff, gsem.at[1])
    i1.wait()
    i2.wait()
    g1 = pltpu.async_copy(acc.at[bloc], gbuf, sem)
    g2 = pltpu.async_copy(cntf_hbm.at[boff], cbuf, ssem.at[0])
    g1.wait()
    w1 = pltpu.async_copy(gbuf, asel_hbm.at[c, pl.ds(b0, BPT)], gsem.at[0])
    g2.wait()
    w2 = pltpu.async_copy(cbuf, csel_hbm.at[c, pl.ds(b0, BPT)], gsem.at[1])
    w1.wait()
    w2.wait()


def _dis_from_cnt(cnt_ref):
    deg = cnt_ref[0, :, 0:1] + cnt_ref[1, :, 0:1] + 1.0
    return lax.rsqrt(deg)


def _dense1_body(x_ref, w_ref, cnt_ref, t_ref):
    u = jnp.dot(x_ref[...], w_ref[...], preferred_element_type=jnp.float32)
    us = u * _dis_from_cnt(cnt_ref)
    t_ref[0] = us[:, :DH]
    t_ref[1] = us[:, DH:]


def _dense2_body(agg_ref, cnt_ref, w_ref, b_ref, t_ref):
    dis = _dis_from_cnt(cnt_ref)
    h0 = jnp.maximum(agg_ref[0] * dis + b_ref[0:1, :DH], 0.0)
    h1 = jnp.maximum(agg_ref[1] * dis + b_ref[0:1, DH:], 0.0)
    u = (jnp.dot(h0, w_ref[:DH, :], preferred_element_type=jnp.float32)
         + jnp.dot(h1, w_ref[DH:, :], preferred_element_type=jnp.float32))
    us = u * dis
    t_ref[0] = us[:, :DH]
    t_ref[1] = us[:, DH:]


def _head_body(asel_ref, csel_ref, b2_ref, wl_ref, bl_ref,
               logp_ref, out_ref, fsel_ref):
    deg = csel_ref[0, :, 0:1] + csel_ref[1, :, 0:1] + 1.0
    dis = lax.rsqrt(deg)
    f0 = jnp.maximum(asel_ref[0] * dis + b2_ref[0:1, :DH], 0.0)
    f1 = jnp.maximum(asel_ref[1] * dis + b2_ref[0:1, DH:], 0.0)
    fsel_ref[:, :DH] = f0
    fsel_ref[:, DH:] = f1
    o = (jnp.dot(f0, wl_ref[:DH, :], preferred_element_type=jnp.float32)
         + jnp.dot(f1, wl_ref[DH:, :], preferred_element_type=jnp.float32))
    o = jnp.maximum(o + bl_ref[0:1, :], 0.0)
    out_ref[...] = o
    m = jnp.max(o, axis=1, keepdims=True)
    lse = m + jnp.log(jnp.sum(jnp.exp(o - m), axis=1, keepdims=True))
    logp_ref[...] = o - lse


_GRID = NPAD // MB

_dense1 = pl.pallas_call(
    _dense1_body,
    grid=(_GRID,),
    in_specs=[
        pl.BlockSpec((MB, D), lambda i: (i, 0)),
        pl.BlockSpec((D, D), lambda i: (0, 0)),
        pl.BlockSpec((NC, MB, 8), lambda i: (0, i, 0)),
    ],
    out_specs=pl.BlockSpec((NC, MB, DH), lambda i: (0, i, 0)),
    out_shape=jax.ShapeDtypeStruct((NC, NPAD, DH), jnp.float32),
)

_dense2 = pl.pallas_call(
    _dense2_body,
    grid=(_GRID,),
    in_specs=[
        pl.BlockSpec((NC, MB, DH), lambda i: (0, i, 0)),
        pl.BlockSpec((NC, MB, 8), lambda i: (0, i, 0)),
        pl.BlockSpec((D, D), lambda i: (0, 0)),
        pl.BlockSpec((1, D), lambda i: (0, 0)),
    ],
    out_specs=pl.BlockSpec((NC, MB, DH), lambda i: (0, i, 0)),
    out_shape=jax.ShapeDtypeStruct((NC, NPAD, DH), jnp.float32),
)

_head = pl.pallas_call(
    _head_body,
    grid=(1,),
    in_specs=[
        pl.BlockSpec((NC, B, DH), lambda i: (0, 0, 0)),
        pl.BlockSpec((NC, B, 8), lambda i: (0, 0, 0)),
        pl.BlockSpec((1, D), lambda i: (0, 0)),
        pl.BlockSpec((D, NCLS), lambda i: (0, 0)),
        pl.BlockSpec((1, NCLS), lambda i: (0, 0)),
    ],
    out_specs=(
        pl.BlockSpec((B, NCLS), lambda i: (0, 0)),
        pl.BlockSpec((B, NCLS), lambda i: (0, 0)),
        pl.BlockSpec((B, D), lambda i: (0, 0)),
    ),
    out_shape=(
        jax.ShapeDtypeStruct((B, NCLS), jnp.float32),
        jax.ShapeDtypeStruct((B, NCLS), jnp.float32),
        jax.ShapeDtypeStruct((B, D), jnp.float32),
    ),
)


def kernel(x, edge_index, batch_index, W1, b1, W2, b2, Wlin, blin):
    src = edge_index[0].astype(jnp.int32)
    dst = edge_index[1].astype(jnp.int32)
    e = src.shape[0]
    pad = EPAD - e
    srcp = jnp.concatenate([src, jnp.zeros((pad,), jnp.int32)])
    dstp = jnp.concatenate([dst, jnp.full((pad,), N, jnp.int32)])
    srcq = srcp.reshape(NCHT, CHUNK)
    dstq_deg = dstp.reshape(NC, NCHT // NC, CHUNK)
    dstq = dstp.reshape(NCHT, CHUNK)
    xp = jnp.concatenate([x, jnp.zeros((NPAD - N, D), jnp.float32)])
    zeros8 = jnp.zeros((NPAD, 8), jnp.float32)
    ones8 = jnp.ones((CHUNK, 8), jnp.float32)

    cnt = _deg_sc(dstq_deg, zeros8, ones8)
    t1 = _dense1(xp, W1, cnt)
    agg1 = _agg_full_sc(t1.reshape(NC * NPAD, DH), srcq, dstq)
    t2 = _dense2(agg1, cnt, W2, b1.reshape(1, D))
    bidx = batch_index.astype(jnp.int32)
    bidx2 = jnp.stack([bidx, bidx + NPAD])
    asel, csel = _agg_batch_sc(t2.reshape(NC * NPAD, DH), srcq, dstq,
                               cnt.reshape(NC * NPAD, 8), bidx2)
    logp, out, fsel = _head(asel, csel, b2.reshape(1, D), Wlin,
                            blin.reshape(1, NCLS))
    return (logp, out, fsel)
